# baseline (device time: 37841 ns/iter reference)
import functools

import jax
import jax.numpy as jnp
from jax import lax
from jax.experimental import pallas as pl
from jax.experimental.pallas import tpu as pltpu

N_DEV = 8
B = 2
SQ = 128
SKV_SH = 128
HQ = 4
DH = 64
HD = HQ * DH
WINDOW = 128
SCALE = 0.125
NEG = -1e9
N_LIVE = 2


def kernel(x, Wq, K_ext, V_ext, Wo):
    k2 = K_ext.reshape(B, SKV_SH, HD)
    v2 = V_ext.reshape(B, SKV_SH, HD)
    d_model = x.shape[-1]

    def body(x_ref, wq_ref, k_ref, v_ref, wo_ref, out_ref,
             kv_k, kv_v, send_sems, recv_sems):
        my = lax.axis_index("i")

        barrier_sem = pltpu.get_barrier_semaphore()
        for nbr in range(N_DEV):
            pl.semaphore_signal(
                barrier_sem, inc=1,
                device_id=(nbr,), device_id_type=pl.DeviceIdType.MESH,
            )
        pl.semaphore_wait(barrier_sem, N_DEV)

        for origin in range(N_LIVE):

            @pl.when(my == origin)
            def _(origin=origin):
                rdmas = []
                for tgt in range(N_DEV):
                    if tgt == origin:
                        continue
                    for kind, (src, buf) in enumerate(
                        ((k_ref, kv_k), (v_ref, kv_v))
                    ):
                        r = pltpu.make_async_remote_copy(
                            src_ref=src,
                            dst_ref=buf.at[origin],
                            send_sem=send_sems.at[tgt, kind],
                            recv_sem=recv_sems.at[origin, kind],
                            device_id=(tgt,),
                            device_id_type=pl.DeviceIdType.MESH,
                        )
                        r.start()
                        rdmas.append(r)
                kv_k[origin] = k_ref[...]
                kv_v[origin] = v_ref[...]
                for r in rdmas:
                    r.wait_send()

        qs = [
            jnp.dot(x_ref[b], wq_ref[...], preferred_element_type=jnp.float32)
            for b in range(B)
        ]

        for origin in range(N_LIVE):

            @pl.when(my != origin)
            def _(origin=origin):
                for kind, (src, buf) in enumerate(
                    ((k_ref, kv_k), (v_ref, kv_v))
                ):
                    r = pltpu.make_async_remote_copy(
                        src_ref=src,
                        dst_ref=buf.at[origin],
                        send_sem=send_sems.at[0, kind],
                        recv_sem=recv_sems.at[origin, kind],
                        device_id=(0,),
                        device_id_type=pl.DeviceIdType.MESH,
                    )
                    r.wait_recv()

        n_keys = N_LIVE * SKV_SH
        qi = lax.broadcasted_iota(jnp.int32, (SQ, n_keys), 0)
        kj = lax.broadcasted_iota(jnp.int32, (SQ, n_keys), 1)
        mask = jnp.abs(qi - kj) <= WINDOW

        for b in range(B):
            kb = jnp.concatenate([kv_k[0, b], kv_k[1, b]], axis=0)
            vb = jnp.concatenate([kv_v[0, b], kv_v[1, b]], axis=0)
            ctx_parts = []
            for h in range(HQ):
                sl = slice(h * DH, (h + 1) * DH)
                qh = qs[b][:, sl]
                kh = kb[:, sl]
                vh = vb[:, sl]
                s = lax.dot_general(
                    qh, kh, (((1,), (1,)), ((), ())),
                    preferred_element_type=jnp.float32,
                ) * SCALE
                s = jnp.where(mask, s, NEG)
                m = jnp.max(s, axis=1, keepdims=True)
                w = jnp.exp(s - m)
                w = w / jnp.sum(w, axis=1, keepdims=True)
                ctx_parts.append(
                    jnp.dot(w, vh, preferred_element_type=jnp.float32)
                )
            ctx = jnp.concatenate(ctx_parts, axis=1)
            out_ref[b] = jnp.dot(
                ctx, wo_ref[...], preferred_element_type=jnp.float32
            )

        @functools.partial(
            pl.run_scoped, second_barrier=pltpu.SemaphoreType.REGULAR
        )
        def _(second_barrier):
            for nbr in range(N_DEV):
                pl.semaphore_signal(
                    second_barrier, inc=1,
                    device_id=(nbr,), device_id_type=pl.DeviceIdType.MESH,
                )
            pl.semaphore_wait(second_barrier, N_DEV)

    return pl.pallas_call(
        body,
        out_shape=jax.ShapeDtypeStruct((B, SQ, d_model), jnp.float32),
        in_specs=[pl.BlockSpec(memory_space=pltpu.VMEM)] * 5,
        out_specs=pl.BlockSpec(memory_space=pltpu.VMEM),
        scratch_shapes=[
            pltpu.VMEM((N_LIVE, B, SKV_SH, HD), jnp.float32),
            pltpu.VMEM((N_LIVE, B, SKV_SH, HD), jnp.float32),
            pltpu.SemaphoreType.DMA((N_DEV, 2)),
            pltpu.SemaphoreType.DMA((N_LIVE, 2)),
        ],
        compiler_params=pltpu.CompilerParams(collective_id=0),
    )(x, Wq, k2, v2, Wo)


# device time: 30559 ns/iter; 1.2383x vs baseline; 1.2383x over previous
import functools

import jax
import jax.numpy as jnp
from jax import lax
from jax.experimental import pallas as pl
from jax.experimental.pallas import tpu as pltpu

N_DEV = 8
B = 2
SQ = 128
SKV_SH = 128
HQ = 4
DH = 64
HD = HQ * DH
WINDOW = 128
SCALE = 0.125
NEG = -1e9
N_LIVE = 2

SEND = {0: (0, [4, 1, 3, 2]), 1: (1, [5, 0, 2, 3])}
FWD = {4: (0, [5, 7, 6]), 5: (1, [4, 6, 7])}
ORIGIN = {0: 0, 1: 1}
RELAY = {0: 4, 1: 5}


def kernel(x, Wq, K_ext, V_ext, Wo):
    k2 = K_ext.reshape(B, SKV_SH, HD)
    v2 = V_ext.reshape(B, SKV_SH, HD)
    d_model = x.shape[-1]

    def body(x_ref, wq_ref, k_ref, v_ref, wo_ref, out_ref,
             kv_k, kv_v, send_sems, recv_sems):
        my = lax.axis_index("i")

        def rdma(src, dst_buf, chunk, kind, tgt):
            return pltpu.make_async_remote_copy(
                src_ref=src,
                dst_ref=dst_buf.at[chunk],
                send_sem=send_sems.at[tgt, kind],
                recv_sem=recv_sems.at[chunk, kind],
                device_id=(tgt,),
                device_id_type=pl.DeviceIdType.MESH,
            )

        def recv_descr(chunk, kind):
            buf = kv_k if kind == 0 else kv_v
            return pltpu.make_async_remote_copy(
                src_ref=k_ref if kind == 0 else v_ref,
                dst_ref=buf.at[chunk],
                send_sem=send_sems.at[0, kind],
                recv_sem=recv_sems.at[chunk, kind],
                device_id=(0,),
                device_id_type=pl.DeviceIdType.MESH,
            )

        barrier_sem = pltpu.get_barrier_semaphore()
        for nbr in range(N_DEV):
            pl.semaphore_signal(
                barrier_sem, inc=1,
                device_id=(nbr,), device_id_type=pl.DeviceIdType.MESH,
            )
        pl.semaphore_wait(barrier_sem, N_DEV)

        for pos, (chunk, targets) in SEND.items():

            @pl.when(my == pos)
            def _(chunk=chunk, targets=targets):
                rdmas = []
                for kind, src in ((0, k_ref), (1, v_ref)):
                    for tgt in targets:
                        r = rdma(src, kv_k if kind == 0 else kv_v,
                                 chunk, kind, tgt)
                        r.start()
                        rdmas.append(r)
                kv_k[chunk] = k_ref[...]
                kv_v[chunk] = v_ref[...]
                for r in rdmas:
                    r.wait_send()

        for pos, (chunk, targets) in FWD.items():

            @pl.when(my == pos)
            def _(chunk=chunk, targets=targets):
                rdmas = []
                for kind in (0, 1):
                    recv_descr(chunk, kind).wait_recv()
                    buf = kv_k if kind == 0 else kv_v
                    for tgt in targets:
                        r = rdma(buf.at[chunk], buf, chunk, kind, tgt)
                        r.start()
                        rdmas.append(r)
                for r in rdmas:
                    r.wait_send()

        qs = [
            jnp.dot(x_ref[b], wq_ref[...], preferred_element_type=jnp.float32)
            for b in range(B)
        ]

        for chunk in range(N_LIVE):

            @pl.when((my != ORIGIN[chunk]) & (my != RELAY[chunk]))
            def _(chunk=chunk):
                for kind in (0, 1):
                    recv_descr(chunk, kind).wait_recv()

        n_keys = N_LIVE * SKV_SH
        qi = lax.broadcasted_iota(jnp.int32, (SQ, n_keys), 0)
        kj = lax.broadcasted_iota(jnp.int32, (SQ, n_keys), 1)
        mask = jnp.abs(qi - kj) <= WINDOW

        for b in range(B):
            kb = jnp.concatenate([kv_k[0, b], kv_k[1, b]], axis=0)
            vb = jnp.concatenate([kv_v[0, b], kv_v[1, b]], axis=0)
            ctx_parts = []
            for h in range(HQ):
                sl = slice(h * DH, (h + 1) * DH)
                qh = qs[b][:, sl]
                kh = kb[:, sl]
                vh = vb[:, sl]
                s = lax.dot_general(
                    qh, kh, (((1,), (1,)), ((), ())),
                    preferred_element_type=jnp.float32,
                ) * SCALE
                s = jnp.where(mask, s, NEG)
                m = jnp.max(s, axis=1, keepdims=True)
                w = jnp.exp(s - m)
                w = w / jnp.sum(w, axis=1, keepdims=True)
                ctx_parts.append(
                    jnp.dot(w, vh, preferred_element_type=jnp.float32)
                )
            ctx = jnp.concatenate(ctx_parts, axis=1)
            out_ref[b] = jnp.dot(
                ctx, wo_ref[...], preferred_element_type=jnp.float32
            )

        @functools.partial(
            pl.run_scoped, second_barrier=pltpu.SemaphoreType.REGULAR
        )
        def _(second_barrier):
            for nbr in range(N_DEV):
                pl.semaphore_signal(
                    second_barrier, inc=1,
                    device_id=(nbr,), device_id_type=pl.DeviceIdType.MESH,
                )
            pl.semaphore_wait(second_barrier, N_DEV)

    return pl.pallas_call(
        body,
        out_shape=jax.ShapeDtypeStruct((B, SQ, d_model), jnp.float32),
        in_specs=[pl.BlockSpec(memory_space=pltpu.VMEM)] * 5,
        out_specs=pl.BlockSpec(memory_space=pltpu.VMEM),
        scratch_shapes=[
            pltpu.VMEM((N_LIVE, B, SKV_SH, HD), jnp.float32),
            pltpu.VMEM((N_LIVE, B, SKV_SH, HD), jnp.float32),
            pltpu.SemaphoreType.DMA((N_DEV, 2)),
            pltpu.SemaphoreType.DMA((N_LIVE, 2)),
        ],
        compiler_params=pltpu.CompilerParams(collective_id=0),
    )(x, Wq, k2, v2, Wo)


# device time: 23098 ns/iter; 1.6383x vs baseline; 1.3230x over previous
import functools

import jax
import jax.numpy as jnp
from jax import lax
from jax.experimental import pallas as pl
from jax.experimental.pallas import tpu as pltpu

N_DEV = 8
B = 2
SQ = 128
SKV_SH = 128
HQ = 4
DH = 64
HD = HQ * DH
WINDOW = 128
SCALE = 0.125
NEG = -1e9
N_LIVE = 2

SEND = {0: (0, [4, 1, 3, 2]), 1: (1, [5, 0, 2, 3])}
FWD = {4: (0, [5, 7, 6]), 5: (1, [4, 6, 7])}
ORIGIN = {0: 0, 1: 1}
RELAY = {0: 4, 1: 5}


def kernel(x, Wq, K_ext, V_ext, Wo):
    k2 = K_ext.reshape(B, SKV_SH, HD).astype(jnp.bfloat16)
    v2 = V_ext.reshape(B, SKV_SH, HD).astype(jnp.bfloat16)
    d_model = x.shape[-1]

    def body(x_ref, wq_ref, k_ref, v_ref, wo_ref, out_ref,
             kv_k, kv_v, send_sems, recv_sems):
        my = lax.axis_index("i")

        def rdma(src, dst_buf, chunk, kind, tgt):
            return pltpu.make_async_remote_copy(
                src_ref=src,
                dst_ref=dst_buf.at[chunk],
                send_sem=send_sems.at[tgt, kind],
                recv_sem=recv_sems.at[chunk, kind],
                device_id=(tgt,),
                device_id_type=pl.DeviceIdType.MESH,
            )

        def recv_descr(chunk, kind):
            buf = kv_k if kind == 0 else kv_v
            return pltpu.make_async_remote_copy(
                src_ref=k_ref if kind == 0 else v_ref,
                dst_ref=buf.at[chunk],
                send_sem=send_sems.at[0, kind],
                recv_sem=recv_sems.at[chunk, kind],
                device_id=(0,),
                device_id_type=pl.DeviceIdType.MESH,
            )

        with jax.named_scope("barrier"):
            barrier_sem = pltpu.get_barrier_semaphore()
            for nbr in range(N_DEV):
                pl.semaphore_signal(
                    barrier_sem, inc=1,
                    device_id=(nbr,), device_id_type=pl.DeviceIdType.MESH,
                )
            pl.semaphore_wait(barrier_sem, N_DEV)

        with jax.named_scope("sends"):
            for pos, (chunk, targets) in SEND.items():

                @pl.when(my == pos)
                def _(chunk=chunk, targets=targets):
                    rdmas = []
                    for kind, src in ((0, k_ref), (1, v_ref)):
                        for tgt in targets:
                            r = rdma(src, kv_k if kind == 0 else kv_v,
                                     chunk, kind, tgt)
                            r.start()
                            rdmas.append(r)
                    kv_k[chunk] = k_ref[...]
                    kv_v[chunk] = v_ref[...]
                    for r in rdmas:
                        r.wait_send()

        with jax.named_scope("relayfwd"):
            for pos, (chunk, targets) in FWD.items():

                @pl.when(my == pos)
                def _(chunk=chunk, targets=targets):
                    rdmas = []
                    for kind in (0, 1):
                        recv_descr(chunk, kind).wait_recv()
                        buf = kv_k if kind == 0 else kv_v
                        for tgt in targets:
                            r = rdma(buf.at[chunk], buf, chunk, kind, tgt)
                            r.start()
                            rdmas.append(r)
                    for r in rdmas:
                        r.wait_send()

        with jax.named_scope("qproj"):
            qs = [
                jnp.dot(x_ref[b], wq_ref[...],
                        preferred_element_type=jnp.float32)
                for b in range(B)
            ]

        with jax.named_scope("recvwait"):
            for chunk in range(N_LIVE):

                @pl.when((my != ORIGIN[chunk]) & (my != RELAY[chunk]))
                def _(chunk=chunk):
                    for kind in (0, 1):
                        recv_descr(chunk, kind).wait_recv()

        n_keys = N_LIVE * SKV_SH
        with jax.named_scope("attn"):
            qi = lax.broadcasted_iota(jnp.int32, (SQ, n_keys), 0)
            kj = lax.broadcasted_iota(jnp.int32, (SQ, n_keys), 1)
            mask = jnp.abs(qi - kj) <= WINDOW

            for b in range(B):
                kb = jnp.concatenate([kv_k[0, b], kv_k[1, b]], axis=0)
                vb = jnp.concatenate([kv_v[0, b], kv_v[1, b]], axis=0)
                ctx_parts = []
                for h in range(HQ):
                    sl = slice(h * DH, (h + 1) * DH)
                    qh = qs[b][:, sl].astype(jnp.bfloat16)
                    kh = kb[:, sl]
                    vh = vb[:, sl]
                    s = lax.dot_general(
                        qh, kh, (((1,), (1,)), ((), ())),
                        preferred_element_type=jnp.float32,
                    ) * SCALE
                    s = jnp.where(mask, s, NEG)
                    m = jnp.max(s, axis=1, keepdims=True)
                    w = jnp.exp(s - m)
                    w = (w / jnp.sum(w, axis=1, keepdims=True)).astype(
                        jnp.bfloat16
                    )
                    ctx_parts.append(
                        jnp.dot(w, vh, preferred_element_type=jnp.float32)
                    )
                ctx = jnp.concatenate(ctx_parts, axis=1)
                out_ref[b] = jnp.dot(
                    ctx, wo_ref[...], preferred_element_type=jnp.float32
                )

        @functools.partial(
            pl.run_scoped, second_barrier=pltpu.SemaphoreType.REGULAR
        )
        def _(second_barrier):
            for nbr in range(N_DEV):
                pl.semaphore_signal(
                    second_barrier, inc=1,
                    device_id=(nbr,), device_id_type=pl.DeviceIdType.MESH,
                )
            pl.semaphore_wait(second_barrier, N_DEV)

    return pl.pallas_call(
        body,
        out_shape=jax.ShapeDtypeStruct((B, SQ, d_model), jnp.float32),
        in_specs=[pl.BlockSpec(memory_space=pltpu.VMEM)] * 5,
        out_specs=pl.BlockSpec(memory_space=pltpu.VMEM),
        scratch_shapes=[
            pltpu.VMEM((N_LIVE, B, SKV_SH, HD), jnp.bfloat16),
            pltpu.VMEM((N_LIVE, B, SKV_SH, HD), jnp.bfloat16),
            pltpu.SemaphoreType.DMA((N_DEV, 2)),
            pltpu.SemaphoreType.DMA((N_LIVE, 2)),
        ],
        compiler_params=pltpu.CompilerParams(collective_id=0),
    )(x, Wq, k2, v2, Wo)
